# baseline (device time: 874323 ns/iter reference)
import jax
import jax.numpy as jnp
from jax import lax
from jax.experimental import pallas as pl
from jax.experimental.pallas import tpu as pltpu

N_DEV = 16
N_EXP = 128
E_LOCAL = 8
CAP = 80
ROWS = E_LOCAL * CAP
D = 512
H = 1024

_sem_signal = getattr(pl, "semaphore_signal", None) or pltpu.semaphore_signal
_sem_wait = getattr(pl, "semaphore_wait", None) or pltpu.semaphore_wait
_DevId = getattr(pl, "DeviceIdType", None) or pltpu.DeviceIdType
_CompilerParams = getattr(pltpu, "CompilerParams", None) or pltpu.TPUCompilerParams


def _prep(x, router_W, route_idx):
    n, d = x.shape
    scores = x @ router_W
    probs = jax.nn.softmax(scores, axis=-1)
    g = jnp.take_along_axis(probs, route_idx, axis=1)
    w = (g / g.sum(axis=1, keepdims=True)).reshape(-1)

    e_flat = route_idx.reshape(-1).astype(jnp.int32)
    P = e_flat.shape[0]
    sort_idx = jnp.argsort(e_flat, stable=True)
    sorted_e = e_flat[sort_idx]
    first = jnp.searchsorted(sorted_e, jnp.arange(N_EXP, dtype=jnp.int32))
    pos_sorted = jnp.arange(P, dtype=jnp.int32) - first[sorted_e].astype(jnp.int32)
    pos = jnp.zeros((P,), jnp.int32).at[sort_idx].set(pos_sorted)
    addr = e_flat * CAP + jnp.minimum(pos, CAP - 1)

    inv = jnp.full((N_DEV * ROWS,), P, jnp.int32).at[addr].set(
        jnp.arange(P, dtype=jnp.int32))
    x_rep = jnp.repeat(x.astype(jnp.bfloat16), 2, axis=0)
    x_pad = jnp.concatenate([x_rep, jnp.zeros((1, d), jnp.bfloat16)], axis=0)
    xs_send = x_pad[inv].reshape(N_DEV, ROWS, d)
    return xs_send, addr, w


def _combine(ys, addr, w, n):
    h = ys.shape[-1]
    contrib = ys.reshape(N_DEV * ROWS, h)[addr].astype(jnp.float32)
    return (contrib * w[:, None].astype(jnp.float32)).reshape(n, 2, h).sum(axis=1)


def _moe_a2a(xs_send, expert_W):

    def body(xs_ref, w_ref, ys_ref, recv_buf, ys_stage,
             send_f, recv_f, send_r, recv_r):
        my = lax.axis_index("i")

        bar = pltpu.get_barrier_semaphore()
        for o in range(1, N_DEV):
            _sem_signal(bar, inc=1, device_id=((my + o) % N_DEV,),
                        device_id_type=_DevId.MESH)
        _sem_wait(bar, N_DEV - 1)

        fwd = []
        for o in range(1, N_DEV):
            dst = (my + o) % N_DEV
            c = pltpu.make_async_remote_copy(
                src_ref=xs_ref.at[dst],
                dst_ref=recv_buf.at[my],
                send_sem=send_f.at[o],
                recv_sem=recv_f.at[my],
                device_id=(dst,),
                device_id_type=_DevId.MESH,
            )
            c.start()
            fwd.append(c)

        recv_buf[my] = xs_ref[my]

        for o in range(N_DEV):
            s = (my + o) % N_DEV
            if o > 0:
                pltpu.make_async_remote_copy(
                    src_ref=recv_buf.at[s],
                    dst_ref=recv_buf.at[s],
                    send_sem=send_f.at[o],
                    recv_sem=recv_f.at[s],
                    device_id=(s,),
                    device_id_type=_DevId.MESH,
                ).wait_recv()
            block = recv_buf[s]
            for e in range(E_LOCAL):
                acc = jnp.dot(block[e * CAP:(e + 1) * CAP, :], w_ref[e],
                              preferred_element_type=jnp.float32)
                ys_stage[0, e * CAP:(e + 1) * CAP, :] = acc.astype(jnp.bfloat16)
            if o == 0:
                ys_ref[s] = ys_stage[0]
            else:
                r = pltpu.make_async_remote_copy(
                    src_ref=ys_stage.at[0],
                    dst_ref=ys_ref.at[my],
                    send_sem=send_r.at[o],
                    recv_sem=recv_r.at[my],
                    device_id=(s,),
                    device_id_type=_DevId.MESH,
                )
                r.start()
                r.wait_send()

        for c in fwd:
            c.wait_send()
        for o in range(1, N_DEV):
            s = (my + o) % N_DEV
            pltpu.make_async_remote_copy(
                src_ref=ys_stage.at[0],
                dst_ref=ys_ref.at[s],
                send_sem=send_r.at[o],
                recv_sem=recv_r.at[s],
                device_id=(s,),
                device_id_type=_DevId.MESH,
            ).wait_recv()

    return pl.pallas_call(
        body,
        out_shape=jax.ShapeDtypeStruct((N_DEV, ROWS, H), jnp.bfloat16),
        in_specs=[pl.BlockSpec(memory_space=pltpu.VMEM),
                  pl.BlockSpec(memory_space=pltpu.VMEM)],
        out_specs=pl.BlockSpec(memory_space=pltpu.VMEM),
        scratch_shapes=[
            pltpu.VMEM((N_DEV, ROWS, D), jnp.bfloat16),
            pltpu.VMEM((1, ROWS, H), jnp.bfloat16),
            pltpu.SemaphoreType.DMA((N_DEV,)),
            pltpu.SemaphoreType.DMA((N_DEV,)),
            pltpu.SemaphoreType.DMA((N_DEV,)),
            pltpu.SemaphoreType.DMA((N_DEV,)),
        ],
        compiler_params=_CompilerParams(collective_id=0),
    )(xs_send, expert_W)


def kernel(x, router_W, route_idx, expert_W):
    xs_send, addr, w = _prep(x, router_W, route_idx)
    ys = _moe_a2a(xs_send, expert_W.astype(jnp.bfloat16))
    return _combine(ys, addr, w, x.shape[0])


# device time: 701809 ns/iter; 1.2458x vs baseline; 1.2458x over previous
import jax
import jax.numpy as jnp
from jax import lax
from jax.experimental import pallas as pl
from jax.experimental.pallas import tpu as pltpu

N_DEV = 16
N_EXP = 128
E_LOCAL = 8
CAP = 80
ROWS = E_LOCAL * CAP
D = 512
H = 1024

_sem_signal = getattr(pl, "semaphore_signal", None) or pltpu.semaphore_signal
_sem_wait = getattr(pl, "semaphore_wait", None) or pltpu.semaphore_wait
_DevId = getattr(pl, "DeviceIdType", None) or pltpu.DeviceIdType
_CompilerParams = getattr(pltpu, "CompilerParams", None) or pltpu.TPUCompilerParams


def _prep(x, router_W, route_idx):
    n, d = x.shape
    scores = x @ router_W
    probs = jax.nn.softmax(scores, axis=-1)
    eids = jnp.arange(N_EXP, dtype=jnp.int32)
    oh2 = route_idx[:, :, None] == eids[None, None, :]
    g = (probs[:, None, :] * oh2).sum(axis=-1)
    w = (g / g.sum(axis=1, keepdims=True)).reshape(-1)

    e_flat = route_idx.reshape(-1).astype(jnp.int32)
    P = e_flat.shape[0]
    oh = (e_flat[:, None] == eids[None, :]).astype(jnp.int32)
    csum = jnp.cumsum(oh, axis=0)
    pos = (csum * oh).sum(axis=1) - 1
    addr = e_flat * CAP + jnp.minimum(pos, CAP - 1)

    inv = jnp.full((N_DEV * ROWS,), P, jnp.int32).at[addr].set(
        jnp.arange(P, dtype=jnp.int32))
    x_rep = jnp.repeat(x.astype(jnp.bfloat16), 2, axis=0)
    x_pad = jnp.concatenate([x_rep, jnp.zeros((1, d), jnp.bfloat16)], axis=0)
    xs_send = x_pad[inv].reshape(N_DEV, ROWS, d)
    return xs_send, addr, w


def _combine(ys, addr, w, n):
    h = ys.shape[-1]
    contrib = ys.reshape(N_DEV * ROWS, h)[addr].astype(jnp.float32)
    return (contrib * w[:, None].astype(jnp.float32)).reshape(n, 2, h).sum(axis=1)


def _moe_a2a(xs_send, expert_W):

    def body(xs_ref, w_ref, ys_ref, recv_buf, ys_stage,
             send_f, recv_f, send_r, recv_r):
        my = lax.axis_index("i")

        bar = pltpu.get_barrier_semaphore()
        for o in range(1, N_DEV):
            _sem_signal(bar, inc=1, device_id=((my + o) % N_DEV,),
                        device_id_type=_DevId.MESH)
        _sem_wait(bar, N_DEV - 1)

        fwd = []
        for o in range(1, N_DEV):
            dst = (my + o) % N_DEV
            c = pltpu.make_async_remote_copy(
                src_ref=xs_ref.at[dst],
                dst_ref=recv_buf.at[my],
                send_sem=send_f.at[o],
                recv_sem=recv_f.at[my],
                device_id=(dst,),
                device_id_type=_DevId.MESH,
            )
            c.start()
            fwd.append(c)

        recv_buf[my] = xs_ref[my]

        ret = [None, None]
        for o in range(N_DEV):
            s = (my + o) % N_DEV
            slot = o % 2
            if o > 0:
                pltpu.make_async_remote_copy(
                    src_ref=recv_buf.at[s],
                    dst_ref=recv_buf.at[s],
                    send_sem=send_f.at[o],
                    recv_sem=recv_f.at[s],
                    device_id=(s,),
                    device_id_type=_DevId.MESH,
                ).wait_recv()
            if ret[slot] is not None:
                ret[slot].wait_send()
                ret[slot] = None
            block = recv_buf[s]
            for e in range(E_LOCAL):
                acc = jnp.dot(block[e * CAP:(e + 1) * CAP, :], w_ref[e],
                              preferred_element_type=jnp.float32)
                ys_stage[slot, e * CAP:(e + 1) * CAP, :] = acc.astype(jnp.bfloat16)
            if o == 0:
                ys_ref[s] = ys_stage[slot]
            else:
                r = pltpu.make_async_remote_copy(
                    src_ref=ys_stage.at[slot],
                    dst_ref=ys_ref.at[my],
                    send_sem=send_r.at[o],
                    recv_sem=recv_r.at[my],
                    device_id=(s,),
                    device_id_type=_DevId.MESH,
                )
                r.start()
                ret[slot] = r

        for r in ret:
            if r is not None:
                r.wait_send()
        for c in fwd:
            c.wait_send()
        for o in range(1, N_DEV):
            s = (my + o) % N_DEV
            pltpu.make_async_remote_copy(
                src_ref=ys_stage.at[0],
                dst_ref=ys_ref.at[s],
                send_sem=send_r.at[o],
                recv_sem=recv_r.at[s],
                device_id=(s,),
                device_id_type=_DevId.MESH,
            ).wait_recv()

    return pl.pallas_call(
        body,
        out_shape=jax.ShapeDtypeStruct((N_DEV, ROWS, H), jnp.bfloat16),
        in_specs=[pl.BlockSpec(memory_space=pltpu.VMEM),
                  pl.BlockSpec(memory_space=pltpu.VMEM)],
        out_specs=pl.BlockSpec(memory_space=pltpu.VMEM),
        scratch_shapes=[
            pltpu.VMEM((N_DEV, ROWS, D), jnp.bfloat16),
            pltpu.VMEM((2, ROWS, H), jnp.bfloat16),
            pltpu.SemaphoreType.DMA((N_DEV,)),
            pltpu.SemaphoreType.DMA((N_DEV,)),
            pltpu.SemaphoreType.DMA((N_DEV,)),
            pltpu.SemaphoreType.DMA((N_DEV,)),
        ],
        compiler_params=_CompilerParams(collective_id=0,
                                        vmem_limit_bytes=64 * 1024 * 1024),
    )(xs_send, expert_W)


def kernel(x, router_W, route_idx, expert_W):
    xs_send, addr, w = _prep(x, router_W, route_idx)
    ys = _moe_a2a(xs_send, expert_W.astype(jnp.bfloat16))
    return _combine(ys, addr, w, x.shape[0])


# device time: 536743 ns/iter; 1.6289x vs baseline; 1.3075x over previous
import jax
import jax.numpy as jnp
from jax import lax
from jax.experimental import pallas as pl
from jax.experimental.pallas import tpu as pltpu

N_DEV = 16
N_EXP = 128
E_LOCAL = 8
CAP = 64
ROWS = E_LOCAL * CAP
D = 512
H = 1024

_sem_signal = getattr(pl, "semaphore_signal", None) or pltpu.semaphore_signal
_sem_wait = getattr(pl, "semaphore_wait", None) or pltpu.semaphore_wait
_DevId = getattr(pl, "DeviceIdType", None) or pltpu.DeviceIdType
_CompilerParams = getattr(pltpu, "CompilerParams", None) or pltpu.TPUCompilerParams


def _prep(x, router_W, route_idx):
    n, d = x.shape
    scores = x @ router_W
    probs = jax.nn.softmax(scores, axis=-1)
    eids = jnp.arange(N_EXP, dtype=jnp.int32)
    oh2 = route_idx[:, :, None] == eids[None, None, :]
    g = (probs[:, None, :] * oh2).sum(axis=-1)
    w = g / g.sum(axis=1, keepdims=True)

    e_flat = route_idx.reshape(-1).astype(jnp.int32)
    P = e_flat.shape[0]
    oh = (e_flat[:, None] == eids[None, :]).astype(jnp.int32)
    csum = jnp.cumsum(oh, axis=0)
    pos = (csum * oh).sum(axis=1) - 1
    addr = e_flat * CAP + jnp.minimum(pos, CAP - 1)

    x_rep = (x[:, None, :] * w[:, :, None]).astype(jnp.bfloat16).reshape(P, d)
    inv = jnp.full((N_DEV * ROWS,), P, jnp.int32).at[addr].set(
        jnp.arange(P, dtype=jnp.int32))
    x_pad = jnp.concatenate([x_rep, jnp.zeros((1, d), jnp.bfloat16)], axis=0)
    xs_send = x_pad[inv].reshape(N_DEV, ROWS, d)
    return xs_send, addr


def _combine(ys, addr, n):
    h = ys.shape[-1]
    ys_flat = ys.reshape(N_DEV * ROWS, h)
    a = addr.reshape(n, 2)
    return (ys_flat[a[:, 0]].astype(jnp.float32)
            + ys_flat[a[:, 1]].astype(jnp.float32))


def _moe_a2a(xs_send, expert_W):

    def body(xs_ref, w_ref, ys_ref, recv_buf, ys_stage,
             send_f, recv_f, send_r, recv_r):
        my = lax.axis_index("i")

        bar = pltpu.get_barrier_semaphore()
        for o in range(1, N_DEV):
            _sem_signal(bar, inc=1, device_id=((my + o) % N_DEV,),
                        device_id_type=_DevId.MESH)
        _sem_wait(bar, N_DEV - 1)

        fwd = []
        for o in range(1, N_DEV):
            dst = (my + o) % N_DEV
            c = pltpu.make_async_remote_copy(
                src_ref=xs_ref.at[dst],
                dst_ref=recv_buf.at[my],
                send_sem=send_f.at[o],
                recv_sem=recv_f.at[my],
                device_id=(dst,),
                device_id_type=_DevId.MESH,
            )
            c.start()
            fwd.append(c)

        recv_buf[my] = xs_ref[my]

        ret = [None, None]
        for o in range(N_DEV):
            s = (my + o) % N_DEV
            slot = o % 2
            if o > 0:
                pltpu.make_async_remote_copy(
                    src_ref=recv_buf.at[s],
                    dst_ref=recv_buf.at[s],
                    send_sem=send_f.at[o],
                    recv_sem=recv_f.at[s],
                    device_id=(s,),
                    device_id_type=_DevId.MESH,
                ).wait_recv()
            if ret[slot] is not None:
                ret[slot].wait_send()
                ret[slot] = None
            block = recv_buf[s]
            for e in range(E_LOCAL):
                acc = jnp.dot(block[e * CAP:(e + 1) * CAP, :], w_ref[e],
                              preferred_element_type=jnp.float32)
                ys_stage[slot, e * CAP:(e + 1) * CAP, :] = acc.astype(jnp.bfloat16)
            if o == 0:
                ys_ref[s] = ys_stage[slot]
            else:
                r = pltpu.make_async_remote_copy(
                    src_ref=ys_stage.at[slot],
                    dst_ref=ys_ref.at[my],
                    send_sem=send_r.at[o],
                    recv_sem=recv_r.at[my],
                    device_id=(s,),
                    device_id_type=_DevId.MESH,
                )
                r.start()
                ret[slot] = r

        for r in ret:
            if r is not None:
                r.wait_send()
        for c in fwd:
            c.wait_send()
        for o in range(1, N_DEV):
            s = (my + o) % N_DEV
            pltpu.make_async_remote_copy(
                src_ref=ys_stage.at[0],
                dst_ref=ys_ref.at[s],
                send_sem=send_r.at[o],
                recv_sem=recv_r.at[s],
                device_id=(s,),
                device_id_type=_DevId.MESH,
            ).wait_recv()

    return pl.pallas_call(
        body,
        out_shape=jax.ShapeDtypeStruct((N_DEV, ROWS, H), jnp.bfloat16),
        in_specs=[pl.BlockSpec(memory_space=pltpu.VMEM),
                  pl.BlockSpec(memory_space=pltpu.VMEM)],
        out_specs=pl.BlockSpec(memory_space=pltpu.VMEM),
        scratch_shapes=[
            pltpu.VMEM((N_DEV, ROWS, D), jnp.bfloat16),
            pltpu.VMEM((2, ROWS, H), jnp.bfloat16),
            pltpu.SemaphoreType.DMA((N_DEV,)),
            pltpu.SemaphoreType.DMA((N_DEV,)),
            pltpu.SemaphoreType.DMA((N_DEV,)),
            pltpu.SemaphoreType.DMA((N_DEV,)),
        ],
        compiler_params=_CompilerParams(collective_id=0,
                                        vmem_limit_bytes=64 * 1024 * 1024),
    )(xs_send, expert_W)


def kernel(x, router_W, route_idx, expert_W):
    xs_send, addr = _prep(x, router_W, route_idx)
    ys = _moe_a2a(xs_send, expert_W.astype(jnp.bfloat16))
    return _combine(ys, addr, x.shape[0])
